# baseline (speedup 1.0000x reference)
import functools

import jax
import jax.numpy as jnp
from jax import lax
from jax.experimental import pallas as pl
from jax.experimental.pallas import tpu as pltpu
from jax.experimental.pallas import tpu_sc as plsc

_N = 883
_NP = 896
_E = 28256
_H = 256
_TILES = 32
_RPT = _NP // _TILES


def _sc_adj_body(src_hbm, dst_hbm, out_hbm, src_v, dst_v, acc_v):
    nc = 2
    wid = lax.axis_index("s") * nc + lax.axis_index("c")
    row0 = wid * _RPT
    pltpu.sync_copy(src_hbm, src_v)
    pltpu.sync_copy(dst_hbm, dst_v)

    zeros16 = jnp.zeros((16,), jnp.float32)
    zun = 8

    def zero_body(i, c):
        base = i * (16 * zun)
        for k in range(zun):
            acc_v[pl.ds(base + 16 * k, 16)] = zeros16
        return c

    lax.fori_loop(0, _RPT * _NP // (16 * zun), zero_body, 0)

    ones16 = jnp.full((16,), 1.0, jnp.float32)

    def scatter16(off):
        d16 = dst_v[pl.ds(off, 16)]
        s16 = src_v[pl.ds(off, 16)]
        rel = d16 - row0
        msk = (rel >= 0) & (rel < _RPT)
        flat = rel * _NP + s16
        plsc.addupdate_scatter(acc_v, [flat], ones16, mask=msk)

    eun = 8
    nmain = _E // (16 * eun)

    def edge_body(i, c):
        base = i * (16 * eun)
        for k in range(eun):
            scatter16(base + 16 * k)
        return c

    lax.fori_loop(0, nmain, edge_body, 0)
    for off in range(nmain * 16 * eun, _E, 16):
        scatter16(off)
    pltpu.sync_copy(acc_v, out_hbm.at[pl.ds(row0 * _NP, _RPT * _NP)])


def _build_adjacency(src, dst):
    mesh = plsc.VectorSubcoreMesh(core_axis_name="c", subcore_axis_name="s")
    fn = pl.kernel(
        _sc_adj_body,
        out_type=jax.ShapeDtypeStruct((_NP * _NP,), jnp.float32),
        mesh=mesh,
        scratch_types=[
            pltpu.VMEM((_E,), jnp.int32),
            pltpu.VMEM((_E,), jnp.int32),
            pltpu.VMEM((_RPT * _NP,), jnp.float32),
        ],
        compiler_params=pltpu.CompilerParams(needs_layout_passes=False),
    )
    return fn(src, dst).reshape(_NP, _NP)


def _tc_body(a_ref, esmw_ref, esmm_ref, sf_ref, mask_ref,
             ws1_ref, ws2_ref, w1e_ref, w1s_ref, b1_ref, w2_ref, b2_ref,
             wd1_ref, wd2_ref, bd_ref, wm2_ref, bm2_ref,
             wp_ref, bp_ref, wdr_ref, bdr_ref, wmg_ref, bmg_ref,
             wc_ref, bc_ref,
             z_ref, probs_ref, cat_ref, dirs_ref, mags_ref, dom_ref,
             an_scr, cs_scr):
    b = pl.program_id(0)

    @pl.when(b == 0)
    def _prep():
        a = a_ref[...]
        deg = jnp.maximum(jnp.sum(a, axis=1, keepdims=True), 1.0)
        a_n = a / deg
        an_scr[...] = a_n
        s1 = jnp.maximum(jnp.dot(sf_ref[...], ws1_ref[...],
                                 preferred_element_type=jnp.float32), 0.0)
        s2 = jnp.maximum(jnp.dot(s1, ws2_ref[...],
                                 preferred_element_type=jnp.float32), 0.0)
        sc = jnp.dot(s2, w1s_ref[...], preferred_element_type=jnp.float32)
        cs_scr[...] = (sc[:, :_H]
                       + jnp.dot(a_n, sc[:, _H:],
                                 preferred_element_type=jnp.float32)
                       + b1_ref[...])

    a_n = an_scr[...]
    c_s = cs_scr[...]

    def run_gnn(esm):
        u = jnp.dot(esm, w1e_ref[...], preferred_element_type=jnp.float32)
        npad = _NP - u.shape[0]
        if npad:
            u = jnp.concatenate(
                [u, jnp.zeros((npad, u.shape[1]), jnp.float32)], axis=0)
        h1 = jnp.maximum(
            u[:, :_H]
            + jnp.dot(a_n, u[:, _H:], preferred_element_type=jnp.float32)
            + c_s, 0.0)
        v = jnp.dot(h1, w2_ref[...], preferred_element_type=jnp.float32)
        h2 = jnp.maximum(
            v[:, :_H]
            + jnp.dot(a_n, v[:, _H:], preferred_element_type=jnp.float32)
            + b2_ref[...], 0.0)
        return h2

    h_wt = run_gnn(esmw_ref[0])
    h_mut = run_gnn(esmm_ref[0])
    colmask = (lax.broadcasted_iota(jnp.int32, (_NP, 1), 0)
               < _N).astype(jnp.float32)
    d = (h_mut - h_wt) * colmask

    mrow = mask_ref[0]
    msum = jnp.maximum(jnp.sum(mrow), 1.0)
    local = jnp.dot(mrow, d, preferred_element_type=jnp.float32) / msum
    onesrow = (lax.broadcasted_iota(jnp.int32, (1, _NP), 1)
               < _N).astype(jnp.float32)
    glob = jnp.dot(onesrow, d, preferred_element_type=jnp.float32) / float(_N)

    zrow = jnp.maximum(
        jnp.dot(local, wd1_ref[...], preferred_element_type=jnp.float32)
        + jnp.dot(glob, wd2_ref[...], preferred_element_type=jnp.float32)
        + bd_ref[...], 0.0)
    z_ref[0] = zrow

    hm2 = jnp.maximum(
        jnp.dot(zrow, wm2_ref[...], preferred_element_type=jnp.float32)
        + bm2_ref[...], 0.0)
    blk = lax.broadcasted_iota(jnp.int32, (1024, 8), 0) // 128
    col = lax.broadcasted_iota(jnp.int32, (1024, 8), 1)
    sel = (blk == col).astype(jnp.float32)

    pr = jnp.dot(hm2 * wp_ref[...], sel,
                 preferred_element_type=jnp.float32) + bp_ref[...]
    probs_ref[0] = 1.0 / (1.0 + jnp.exp(-pr))
    dirs_ref[0] = jnp.dot(hm2 * wdr_ref[...], sel,
                          preferred_element_type=jnp.float32) + bdr_ref[...]
    mg = jnp.dot(hm2 * wmg_ref[...], sel,
                 preferred_element_type=jnp.float32) + bmg_ref[...]
    mags_ref[0] = jnp.maximum(mg, 0.0) + jnp.log1p(jnp.exp(-jnp.abs(mg)))

    cat = jnp.dot(zrow, wc_ref[...],
                  preferred_element_type=jnp.float32) + bc_ref[...]
    cat_ref[0] = cat
    mx = jnp.max(cat, axis=1, keepdims=True)
    idx = lax.broadcasted_iota(jnp.int32, (1, 8), 1)
    dom_ref[0] = jnp.min(jnp.where(cat >= mx, idx, jnp.int32(2**30)),
                         axis=1, keepdims=True)


def _full(shape):
    nd = len(shape)
    return pl.BlockSpec(shape, lambda b, _nd=nd: (0,) * _nd)


def _tc_call(adj, esm_wt, esm_mut, sfp, maskp, ws1, ws2, w1e, w1s, b1, w2,
             b2, wd1, wd2, bd, wm2, bm2, wp, bp, wdr, bdr, wmg, bmg, wc, bc):
    B = esm_wt.shape[0]
    n = esm_wt.shape[1]
    grid = (B,)
    in_specs = [
        _full((_NP, _NP)),
        pl.BlockSpec((1, n, 1280), lambda b: (b, 0, 0)),
        pl.BlockSpec((1, n, 1280), lambda b: (b, 0, 0)),
        _full(sfp.shape),
        pl.BlockSpec((1, 1, _NP), lambda b: (b, 0, 0)),
    ] + [_full(w.shape) for w in
         (ws1, ws2, w1e, w1s, b1, w2, b2, wd1, wd2, bd, wm2, bm2,
          wp, bp, wdr, bdr, wmg, bmg, wc, bc)]
    out_shape = (
        jax.ShapeDtypeStruct((B, 1, _H), jnp.float32),
        jax.ShapeDtypeStruct((B, 1, 8), jnp.float32),
        jax.ShapeDtypeStruct((B, 1, 8), jnp.float32),
        jax.ShapeDtypeStruct((B, 1, 8), jnp.float32),
        jax.ShapeDtypeStruct((B, 1, 8), jnp.float32),
        jax.ShapeDtypeStruct((B, 1, 1), jnp.int32),
    )
    out_specs = (
        pl.BlockSpec((1, 1, _H), lambda b: (b, 0, 0)),
        pl.BlockSpec((1, 1, 8), lambda b: (b, 0, 0)),
        pl.BlockSpec((1, 1, 8), lambda b: (b, 0, 0)),
        pl.BlockSpec((1, 1, 8), lambda b: (b, 0, 0)),
        pl.BlockSpec((1, 1, 8), lambda b: (b, 0, 0)),
        pl.BlockSpec((1, 1, 1), lambda b: (b, 0, 0)),
    )
    return pl.pallas_call(
        _tc_body,
        grid=grid,
        in_specs=in_specs,
        out_specs=out_specs,
        out_shape=out_shape,
        scratch_shapes=[
            pltpu.VMEM((_NP, _NP), jnp.float32),
            pltpu.VMEM((_NP, _H), jnp.float32),
        ],
        compiler_params=pltpu.CompilerParams(
            dimension_semantics=("arbitrary",)),
    )(adj, esm_wt, esm_mut, sfp, maskp, ws1, ws2, w1e, w1s, b1, w2, b2,
      wd1, wd2, bd, wm2, bm2, wp, bp, wdr, bdr, wmg, bmg, wc, bc)


def kernel(esm_wt, esm_mut, struct_feat, edge_index, mutation_mask,
           W_s1, b_s1, W_s2, b_s2, W_g1, b_g1, W_g2, b_g2, W_d, b_d,
           W_m, b_m, w_prob, b_prob, w_dir, b_dir, w_mag, b_mag, W_c, b_c):
    B = esm_wt.shape[0]
    pad = _NP - _N

    adj = _build_adjacency(edge_index[0], edge_index[1])

    sfp = jnp.pad(struct_feat, ((0, pad), (0, 0)))
    maskp = jnp.pad(mutation_mask, ((0, 0), (0, pad))).reshape(B, 1, _NP)

    din = 1280 + 128
    w1e = jnp.concatenate([W_g1[0:1280], W_g1[din:din + 1280]], axis=1)
    w1s = jnp.concatenate([W_g1[1280:din], W_g1[din + 1280:2 * din]], axis=1)
    w2 = jnp.concatenate([W_g2[0:_H], W_g2[_H:2 * _H]], axis=1)
    wd1, wd2 = W_d[:_H], W_d[_H:]
    wm2 = jnp.transpose(W_m, (1, 0, 2)).reshape(_H, 1024)

    r2 = lambda x: x.reshape(1, -1)
    outs = _tc_call(
        adj, esm_wt, esm_mut, sfp, maskp,
        W_s1, W_s2, w1e, w1s, r2(b_g1), w2, r2(b_g2),
        wd1, wd2, r2(b_d), wm2, r2(b_m),
        r2(w_prob), r2(b_prob), r2(w_dir), r2(b_dir), r2(w_mag), r2(b_mag),
        W_c, r2(b_c))
    z, probs, cat, dirs, mags, dom = outs
    return (z.reshape(B, _H), probs.reshape(B, 8), cat.reshape(B, 8),
            dirs.reshape(B, 8), mags.reshape(B, 8), dom.reshape(B))


# speedup vs baseline: 1.0128x; 1.0128x over previous
import functools

import jax
import jax.numpy as jnp
from jax import lax
from jax.experimental import pallas as pl
from jax.experimental.pallas import tpu as pltpu
from jax.experimental.pallas import tpu_sc as plsc

_N = 883
_NP = 896
_E = 28256
_H = 256
_TILES = 32
_RPT = _NP // _TILES


def _sc_adj_body(src_hbm, dst_hbm, out_hbm, src_v, dst_v, acc_v):
    nc = 2
    wid = lax.axis_index("s") * nc + lax.axis_index("c")
    row0 = wid * _RPT
    pltpu.sync_copy(src_hbm, src_v)
    pltpu.sync_copy(dst_hbm, dst_v)

    zeros16 = jnp.zeros((16,), jnp.float32)
    zun = 8

    def zero_body(i, c):
        base = i * (16 * zun)
        for k in range(zun):
            acc_v[pl.ds(base + 16 * k, 16)] = zeros16
        return c

    lax.fori_loop(0, _RPT * _NP // (16 * zun), zero_body, 0)

    ones16 = jnp.full((16,), 1.0, jnp.float32)

    def scatter16(off):
        d16 = dst_v[pl.ds(off, 16)]
        s16 = src_v[pl.ds(off, 16)]
        rel = d16 - row0
        msk = (rel >= 0) & (rel < _RPT)
        flat = rel * _NP + s16
        plsc.addupdate_scatter(acc_v, [flat], ones16, mask=msk)

    eun = 8
    nmain = _E // (16 * eun)

    def edge_body(i, c):
        base = i * (16 * eun)
        for k in range(eun):
            scatter16(base + 16 * k)
        return c

    lax.fori_loop(0, nmain, edge_body, 0)
    for off in range(nmain * 16 * eun, _E, 16):
        scatter16(off)
    pltpu.sync_copy(acc_v, out_hbm.at[pl.ds(row0 * _NP, _RPT * _NP)])


def _build_adjacency(src, dst):
    mesh = plsc.VectorSubcoreMesh(core_axis_name="c", subcore_axis_name="s")
    fn = pl.kernel(
        _sc_adj_body,
        out_type=jax.ShapeDtypeStruct((_NP * _NP,), jnp.float32),
        mesh=mesh,
        scratch_types=[
            pltpu.VMEM((_E,), jnp.int32),
            pltpu.VMEM((_E,), jnp.int32),
            pltpu.VMEM((_RPT * _NP,), jnp.float32),
        ],
        compiler_params=pltpu.CompilerParams(needs_layout_passes=False),
    )
    return fn(src, dst).reshape(_NP, _NP)


def _tc_body(a_ref, esmw_ref, esmm_ref, sf_ref, mask_ref,
             ws1_ref, ws2_ref, w1e_ref, w1s_ref, b1_ref, w2_ref, b2_ref,
             wd1_ref, wd2_ref, bd_ref, wm2_ref, bm2_ref,
             wp_ref, bp_ref, wdr_ref, bdr_ref, wmg_ref, bmg_ref,
             wc_ref, bc_ref,
             z_ref, probs_ref, cat_ref, dirs_ref, mags_ref, dom_ref,
             an_scr, cs_scr):
    b = pl.program_id(0)

    @pl.when(b == 0)
    def _prep():
        a = a_ref[...]
        deg = jnp.maximum(jnp.sum(a, axis=1, keepdims=True), 1.0)
        a_n = a / deg
        an_scr[...] = a_n
        s1 = jnp.maximum(jnp.dot(sf_ref[...], ws1_ref[...],
                                 preferred_element_type=jnp.float32), 0.0)
        s2 = jnp.maximum(jnp.dot(s1, ws2_ref[...],
                                 preferred_element_type=jnp.float32), 0.0)
        sc = jnp.dot(s2, w1s_ref[...], preferred_element_type=jnp.float32)
        cs_scr[...] = (sc[:, :_H]
                       + jnp.dot(a_n, sc[:, _H:],
                                 preferred_element_type=jnp.float32)
                       + b1_ref[...])

    a_n = an_scr[...]
    c_s = cs_scr[...]

    def run_gnn(esm):
        u = jnp.dot(esm.astype(jnp.bfloat16), w1e_ref[...],
                    preferred_element_type=jnp.float32)
        npad = _NP - u.shape[0]
        if npad:
            u = jnp.concatenate(
                [u, jnp.zeros((npad, u.shape[1]), jnp.float32)], axis=0)
        h1 = jnp.maximum(
            u[:, :_H]
            + jnp.dot(a_n, u[:, _H:], preferred_element_type=jnp.float32)
            + c_s, 0.0)
        v = jnp.dot(h1, w2_ref[...], preferred_element_type=jnp.float32)
        h2 = jnp.maximum(
            v[:, :_H]
            + jnp.dot(a_n, v[:, _H:], preferred_element_type=jnp.float32)
            + b2_ref[...], 0.0)
        return h2

    h_wt = run_gnn(esmw_ref[0])
    h_mut = run_gnn(esmm_ref[0])
    colmask = (lax.broadcasted_iota(jnp.int32, (_NP, 1), 0)
               < _N).astype(jnp.float32)
    d = (h_mut - h_wt) * colmask

    mrow = mask_ref[0]
    msum = jnp.maximum(jnp.sum(mrow), 1.0)
    local = jnp.dot(mrow, d, preferred_element_type=jnp.float32) / msum
    onesrow = (lax.broadcasted_iota(jnp.int32, (1, _NP), 1)
               < _N).astype(jnp.float32)
    glob = jnp.dot(onesrow, d, preferred_element_type=jnp.float32) / float(_N)

    zrow = jnp.maximum(
        jnp.dot(local, wd1_ref[...], preferred_element_type=jnp.float32)
        + jnp.dot(glob, wd2_ref[...], preferred_element_type=jnp.float32)
        + bd_ref[...], 0.0)
    z_ref[0] = zrow

    hm2 = jnp.maximum(
        jnp.dot(zrow, wm2_ref[...], preferred_element_type=jnp.float32)
        + bm2_ref[...], 0.0)
    blk = lax.broadcasted_iota(jnp.int32, (1024, 8), 0) // 128
    col = lax.broadcasted_iota(jnp.int32, (1024, 8), 1)
    sel = (blk == col).astype(jnp.float32)

    pr = jnp.dot(hm2 * wp_ref[...], sel,
                 preferred_element_type=jnp.float32) + bp_ref[...]
    probs_ref[0] = 1.0 / (1.0 + jnp.exp(-pr))
    dirs_ref[0] = jnp.dot(hm2 * wdr_ref[...], sel,
                          preferred_element_type=jnp.float32) + bdr_ref[...]
    mg = jnp.dot(hm2 * wmg_ref[...], sel,
                 preferred_element_type=jnp.float32) + bmg_ref[...]
    mags_ref[0] = jnp.maximum(mg, 0.0) + jnp.log1p(jnp.exp(-jnp.abs(mg)))

    cat = jnp.dot(zrow, wc_ref[...],
                  preferred_element_type=jnp.float32) + bc_ref[...]
    cat_ref[0] = cat
    mx = jnp.max(cat, axis=1, keepdims=True)
    idx = lax.broadcasted_iota(jnp.int32, (1, 8), 1)
    dom_ref[0] = jnp.min(jnp.where(cat >= mx, idx, jnp.int32(2**30)),
                         axis=1, keepdims=True)


def _full(shape):
    nd = len(shape)
    return pl.BlockSpec(shape, lambda b, _nd=nd: (0,) * _nd)


def _tc_call(adj, esm_wt, esm_mut, sfp, maskp, ws1, ws2, w1e, w1s, b1, w2,
             b2, wd1, wd2, bd, wm2, bm2, wp, bp, wdr, bdr, wmg, bmg, wc, bc):
    B = esm_wt.shape[0]
    n = esm_wt.shape[1]
    grid = (B,)
    in_specs = [
        _full((_NP, _NP)),
        pl.BlockSpec((1, n, 1280), lambda b: (b, 0, 0)),
        pl.BlockSpec((1, n, 1280), lambda b: (b, 0, 0)),
        _full(sfp.shape),
        pl.BlockSpec((1, 1, _NP), lambda b: (b, 0, 0)),
    ] + [_full(w.shape) for w in
         (ws1, ws2, w1e, w1s, b1, w2, b2, wd1, wd2, bd, wm2, bm2,
          wp, bp, wdr, bdr, wmg, bmg, wc, bc)]
    out_shape = (
        jax.ShapeDtypeStruct((B, 1, _H), jnp.float32),
        jax.ShapeDtypeStruct((B, 1, 8), jnp.float32),
        jax.ShapeDtypeStruct((B, 1, 8), jnp.float32),
        jax.ShapeDtypeStruct((B, 1, 8), jnp.float32),
        jax.ShapeDtypeStruct((B, 1, 8), jnp.float32),
        jax.ShapeDtypeStruct((B, 1, 1), jnp.int32),
    )
    out_specs = (
        pl.BlockSpec((1, 1, _H), lambda b: (b, 0, 0)),
        pl.BlockSpec((1, 1, 8), lambda b: (b, 0, 0)),
        pl.BlockSpec((1, 1, 8), lambda b: (b, 0, 0)),
        pl.BlockSpec((1, 1, 8), lambda b: (b, 0, 0)),
        pl.BlockSpec((1, 1, 8), lambda b: (b, 0, 0)),
        pl.BlockSpec((1, 1, 1), lambda b: (b, 0, 0)),
    )
    return pl.pallas_call(
        _tc_body,
        grid=grid,
        in_specs=in_specs,
        out_specs=out_specs,
        out_shape=out_shape,
        scratch_shapes=[
            pltpu.VMEM((_NP, _NP), jnp.float32),
            pltpu.VMEM((_NP, _H), jnp.float32),
        ],
        compiler_params=pltpu.CompilerParams(
            dimension_semantics=("arbitrary",)),
    )(adj, esm_wt, esm_mut, sfp, maskp, ws1, ws2, w1e, w1s, b1, w2, b2,
      wd1, wd2, bd, wm2, bm2, wp, bp, wdr, bdr, wmg, bmg, wc, bc)


def kernel(esm_wt, esm_mut, struct_feat, edge_index, mutation_mask,
           W_s1, b_s1, W_s2, b_s2, W_g1, b_g1, W_g2, b_g2, W_d, b_d,
           W_m, b_m, w_prob, b_prob, w_dir, b_dir, w_mag, b_mag, W_c, b_c):
    B = esm_wt.shape[0]
    pad = _NP - _N

    adj = _build_adjacency(edge_index[0], edge_index[1])

    sfp = jnp.pad(struct_feat, ((0, pad), (0, 0)))
    maskp = jnp.pad(mutation_mask, ((0, 0), (0, pad))).reshape(B, 1, _NP)

    din = 1280 + 128
    w1e = jnp.concatenate([W_g1[0:1280], W_g1[din:din + 1280]],
                          axis=1).astype(jnp.bfloat16)
    w1s = jnp.concatenate([W_g1[1280:din], W_g1[din + 1280:2 * din]], axis=1)
    w2 = jnp.concatenate([W_g2[0:_H], W_g2[_H:2 * _H]], axis=1)
    wd1, wd2 = W_d[:_H], W_d[_H:]
    wm2 = jnp.transpose(W_m, (1, 0, 2)).reshape(_H, 1024)

    r2 = lambda x: x.reshape(1, -1)
    outs = _tc_call(
        adj, esm_wt, esm_mut, sfp, maskp,
        W_s1, W_s2, w1e, w1s, r2(b_g1), w2, r2(b_g2),
        wd1, wd2, r2(b_d), wm2, r2(b_m),
        r2(w_prob), r2(b_prob), r2(w_dir), r2(b_dir), r2(w_mag), r2(b_mag),
        W_c, r2(b_c))
    z, probs, cat, dirs, mags, dom = outs
    return (z.reshape(B, _H), probs.reshape(B, 8), cat.reshape(B, 8),
            dirs.reshape(B, 8), mags.reshape(B, 8), dom.reshape(B))


# speedup vs baseline: 1.0706x; 1.0571x over previous
import functools

import jax
import jax.numpy as jnp
from jax import lax
from jax.experimental import pallas as pl
from jax.experimental.pallas import tpu as pltpu
from jax.experimental.pallas import tpu_sc as plsc

_N = 883
_NP = 896
_E = 28256
_H = 256
_TILES = 32
_RPT = _NP // 16


def _sc_adj_body(src_hbm, dst_hbm, out_hbm, src_v, dst_v, acc_v):
    wid = lax.axis_index("s")
    row0 = wid * _RPT
    pltpu.sync_copy(src_hbm, src_v)
    pltpu.sync_copy(dst_hbm, dst_v)

    zeros16 = jnp.zeros((16,), jnp.float32)
    zun = 8

    def zero_body(i, c):
        base = i * (16 * zun)
        for k in range(zun):
            acc_v[pl.ds(base + 16 * k, 16)] = zeros16
        return c

    lax.fori_loop(0, _RPT * _NP // (16 * zun), zero_body, 0)

    ones16 = jnp.full((16,), 1.0, jnp.float32)

    def scatter16(off):
        d16 = dst_v[pl.ds(off, 16)]
        s16 = src_v[pl.ds(off, 16)]
        rel = d16 - row0
        msk = (rel >= 0) & (rel < _RPT)
        flat = rel * _NP + s16
        plsc.addupdate_scatter(acc_v, [flat], ones16, mask=msk)

    eun = 8
    nmain = _E // (16 * eun)

    def edge_body(i, c):
        base = i * (16 * eun)
        for k in range(eun):
            scatter16(base + 16 * k)
        return c

    lax.fori_loop(0, nmain, edge_body, 0)
    for off in range(nmain * 16 * eun, _E, 16):
        scatter16(off)
    pltpu.sync_copy(acc_v, out_hbm.at[pl.ds(row0 * _NP, _RPT * _NP)])


def _build_adjacency(src, dst):
    mesh = plsc.VectorSubcoreMesh(core_axis_name="c", subcore_axis_name="s", num_cores=1)
    fn = pl.kernel(
        _sc_adj_body,
        out_type=jax.ShapeDtypeStruct((_NP * _NP,), jnp.float32),
        mesh=mesh,
        scratch_types=[
            pltpu.VMEM((_E,), jnp.int32),
            pltpu.VMEM((_E,), jnp.int32),
            pltpu.VMEM((_RPT * _NP,), jnp.float32),
        ],
        compiler_params=pltpu.CompilerParams(needs_layout_passes=False),
    )
    return fn(src, dst).reshape(_NP, _NP)


def _tc_body(a_ref, esmw_ref, esmm_ref, sf_ref, mask_ref,
             ws1_ref, ws2_ref, w1e_ref, w1s_ref, b1_ref, w2_ref, b2_ref,
             wd1_ref, wd2_ref, bd_ref, wm2_ref, bm2_ref,
             wp_ref, bp_ref, wdr_ref, bdr_ref, wmg_ref, bmg_ref,
             wc_ref, bc_ref,
             z_ref, probs_ref, cat_ref, dirs_ref, mags_ref, dom_ref,
             an_scr, cs_scr):
    b = pl.program_id(0)

    @pl.when(b == 0)
    def _prep():
        a = a_ref[...]
        deg = jnp.maximum(jnp.sum(a, axis=1, keepdims=True), 1.0)
        a_n = a / deg
        an_scr[...] = a_n
        s1 = jnp.maximum(jnp.dot(sf_ref[...], ws1_ref[...],
                                 preferred_element_type=jnp.float32), 0.0)
        s2 = jnp.maximum(jnp.dot(s1, ws2_ref[...],
                                 preferred_element_type=jnp.float32), 0.0)
        sc = jnp.dot(s2, w1s_ref[...], preferred_element_type=jnp.float32)
        cs_scr[...] = (sc[:, :_H]
                       + jnp.dot(a_n, sc[:, _H:],
                                 preferred_element_type=jnp.float32)
                       + b1_ref[...])

    a_n = an_scr[...]
    c_s = cs_scr[...]

    def run_gnn(esm):
        u = jnp.dot(esm.astype(jnp.bfloat16), w1e_ref[...],
                    preferred_element_type=jnp.float32)
        npad = _NP - u.shape[0]
        if npad:
            u = jnp.concatenate(
                [u, jnp.zeros((npad, u.shape[1]), jnp.float32)], axis=0)
        h1 = jnp.maximum(
            u[:, :_H]
            + jnp.dot(a_n, u[:, _H:], preferred_element_type=jnp.float32)
            + c_s, 0.0)
        v = jnp.dot(h1, w2_ref[...], preferred_element_type=jnp.float32)
        h2 = jnp.maximum(
            v[:, :_H]
            + jnp.dot(a_n, v[:, _H:], preferred_element_type=jnp.float32)
            + b2_ref[...], 0.0)
        return h2

    h_wt = run_gnn(esmw_ref[0])
    h_mut = run_gnn(esmm_ref[0])
    colmask = (lax.broadcasted_iota(jnp.int32, (_NP, 1), 0)
               < _N).astype(jnp.float32)
    d = (h_mut - h_wt) * colmask

    mrow = mask_ref[0]
    msum = jnp.maximum(jnp.sum(mrow), 1.0)
    local = jnp.dot(mrow, d, preferred_element_type=jnp.float32) / msum
    onesrow = (lax.broadcasted_iota(jnp.int32, (1, _NP), 1)
               < _N).astype(jnp.float32)
    glob = jnp.dot(onesrow, d, preferred_element_type=jnp.float32) / float(_N)

    zrow = jnp.maximum(
        jnp.dot(local, wd1_ref[...], preferred_element_type=jnp.float32)
        + jnp.dot(glob, wd2_ref[...], preferred_element_type=jnp.float32)
        + bd_ref[...], 0.0)
    z_ref[0] = zrow

    hm2 = jnp.maximum(
        jnp.dot(zrow, wm2_ref[...], preferred_element_type=jnp.float32)
        + bm2_ref[...], 0.0)
    blk = lax.broadcasted_iota(jnp.int32, (1024, 8), 0) // 128
    col = lax.broadcasted_iota(jnp.int32, (1024, 8), 1)
    sel = (blk == col).astype(jnp.float32)

    pr = jnp.dot(hm2 * wp_ref[...], sel,
                 preferred_element_type=jnp.float32) + bp_ref[...]
    probs_ref[0] = 1.0 / (1.0 + jnp.exp(-pr))
    dirs_ref[0] = jnp.dot(hm2 * wdr_ref[...], sel,
                          preferred_element_type=jnp.float32) + bdr_ref[...]
    mg = jnp.dot(hm2 * wmg_ref[...], sel,
                 preferred_element_type=jnp.float32) + bmg_ref[...]
    mags_ref[0] = jnp.maximum(mg, 0.0) + jnp.log1p(jnp.exp(-jnp.abs(mg)))

    cat = jnp.dot(zrow, wc_ref[...],
                  preferred_element_type=jnp.float32) + bc_ref[...]
    cat_ref[0] = cat
    mx = jnp.max(cat, axis=1, keepdims=True)
    idx = lax.broadcasted_iota(jnp.int32, (1, 8), 1)
    dom_ref[0] = jnp.min(jnp.where(cat >= mx, idx, jnp.int32(2**30)),
                         axis=1, keepdims=True)


def _full(shape):
    nd = len(shape)
    return pl.BlockSpec(shape, lambda b, _nd=nd: (0,) * _nd)


def _tc_call(adj, esm_wt, esm_mut, sfp, maskp, ws1, ws2, w1e, w1s, b1, w2,
             b2, wd1, wd2, bd, wm2, bm2, wp, bp, wdr, bdr, wmg, bmg, wc, bc):
    B = esm_wt.shape[0]
    n = esm_wt.shape[1]
    grid = (B,)
    in_specs = [
        _full((_NP, _NP)),
        pl.BlockSpec((1, n, 1280), lambda b: (b, 0, 0)),
        pl.BlockSpec((1, n, 1280), lambda b: (b, 0, 0)),
        _full(sfp.shape),
        pl.BlockSpec((1, 1, _NP), lambda b: (b, 0, 0)),
    ] + [_full(w.shape) for w in
         (ws1, ws2, w1e, w1s, b1, w2, b2, wd1, wd2, bd, wm2, bm2,
          wp, bp, wdr, bdr, wmg, bmg, wc, bc)]
    out_shape = (
        jax.ShapeDtypeStruct((B, 1, _H), jnp.float32),
        jax.ShapeDtypeStruct((B, 1, 8), jnp.float32),
        jax.ShapeDtypeStruct((B, 1, 8), jnp.float32),
        jax.ShapeDtypeStruct((B, 1, 8), jnp.float32),
        jax.ShapeDtypeStruct((B, 1, 8), jnp.float32),
        jax.ShapeDtypeStruct((B, 1, 1), jnp.int32),
    )
    out_specs = (
        pl.BlockSpec((1, 1, _H), lambda b: (b, 0, 0)),
        pl.BlockSpec((1, 1, 8), lambda b: (b, 0, 0)),
        pl.BlockSpec((1, 1, 8), lambda b: (b, 0, 0)),
        pl.BlockSpec((1, 1, 8), lambda b: (b, 0, 0)),
        pl.BlockSpec((1, 1, 8), lambda b: (b, 0, 0)),
        pl.BlockSpec((1, 1, 1), lambda b: (b, 0, 0)),
    )
    return pl.pallas_call(
        _tc_body,
        grid=grid,
        in_specs=in_specs,
        out_specs=out_specs,
        out_shape=out_shape,
        scratch_shapes=[
            pltpu.VMEM((_NP, _NP), jnp.float32),
            pltpu.VMEM((_NP, _H), jnp.float32),
        ],
        compiler_params=pltpu.CompilerParams(
            dimension_semantics=("arbitrary",)),
    )(adj, esm_wt, esm_mut, sfp, maskp, ws1, ws2, w1e, w1s, b1, w2, b2,
      wd1, wd2, bd, wm2, bm2, wp, bp, wdr, bdr, wmg, bmg, wc, bc)


def kernel(esm_wt, esm_mut, struct_feat, edge_index, mutation_mask,
           W_s1, b_s1, W_s2, b_s2, W_g1, b_g1, W_g2, b_g2, W_d, b_d,
           W_m, b_m, w_prob, b_prob, w_dir, b_dir, w_mag, b_mag, W_c, b_c):
    B = esm_wt.shape[0]
    pad = _NP - _N

    adj = _build_adjacency(edge_index[0], edge_index[1])

    sfp = jnp.pad(struct_feat, ((0, pad), (0, 0)))
    maskp = jnp.pad(mutation_mask, ((0, 0), (0, pad))).reshape(B, 1, _NP)

    din = 1280 + 128
    w1e = jnp.concatenate([W_g1[0:1280], W_g1[din:din + 1280]],
                          axis=1).astype(jnp.bfloat16)
    w1s = jnp.concatenate([W_g1[1280:din], W_g1[din + 1280:2 * din]], axis=1)
    w2 = jnp.concatenate([W_g2[0:_H], W_g2[_H:2 * _H]], axis=1)
    wd1, wd2 = W_d[:_H], W_d[_H:]
    wm2 = jnp.transpose(W_m, (1, 0, 2)).reshape(_H, 1024)

    r2 = lambda x: x.reshape(1, -1)
    outs = _tc_call(
        adj, esm_wt, esm_mut, sfp, maskp,
        W_s1, W_s2, w1e, w1s, r2(b_g1), w2, r2(b_g2),
        wd1, wd2, r2(b_d), wm2, r2(b_m),
        r2(w_prob), r2(b_prob), r2(w_dir), r2(b_dir), r2(w_mag), r2(b_mag),
        W_c, r2(b_c))
    z, probs, cat, dirs, mags, dom = outs
    return (z.reshape(B, _H), probs.reshape(B, 8), cat.reshape(B, 8),
            dirs.reshape(B, 8), mags.reshape(B, 8), dom.reshape(B))
